# baseline (speedup 1.0000x reference)
import jax
import jax.numpy as jnp
import numpy as np
from jax import lax
from jax.experimental import pallas as pl
from jax.experimental.pallas import tpu as pltpu
from jax.experimental.pallas import tpu_sc as plsc

_rng = np.random.default_rng(0)
_DRAWS = [int(_rng.integers(low=0, high=15)) for _ in range(4)]
_D0 = _DRAWS[0]
_K = 8.0

_ROWS = 1024
_HSPLIT = 12
_LANES = 16
_GRP = 128


def _argmax_copy_core(in_ref, out_ref, cand_ref):
    rb = pl.program_id(1)
    x = in_ref[0]
    out_ref[...] = in_ref[...]
    m = jnp.max(x, axis=1, keepdims=True)
    src = x.shape[1]
    iota = lax.broadcasted_iota(jnp.int32, x.shape, 1)
    idx = jnp.min(jnp.where(x == m, iota, src), axis=1)
    cand_ref[0, 0, pl.ds(rb * _ROWS, _ROWS)] = idx


def _argmax_copy_body_a(in_ref, out_ref, cand_ref):
    _argmax_copy_core(in_ref, out_ref, cand_ref)


def _argmax_copy_body_b(in_ref, buf_ref, out_ref, cand_ref):
    del buf_ref
    _argmax_copy_core(in_ref, out_ref, cand_ref)


def _fixup_body(cond_ref, buf_ref, out_ref, sem):
    del buf_ref

    @pl.when(cond_ref[0] == 1)
    def _():
        for h in range(out_ref.shape[0]):
            if h != _D0:
                cp = pltpu.make_async_copy(out_ref.at[_D0], out_ref.at[h], sem)
                cp.start()
                cp.wait()


def _memb_phase(cand_hbm, cand_v, memb_v, sh_memb, s):
    src = memb_v.shape[0]
    chunks = src // _LANES
    zeros = jnp.zeros((_LANES,), jnp.float32)
    ones = jnp.ones((_LANES,), jnp.float32)
    pltpu.sync_copy(cand_hbm.at[s], cand_v)

    def zbody(i, _):
        memb_v[pl.ds(i * _LANES, _LANES)] = zeros
        return 0

    lax.fori_loop(0, chunks, zbody, 0)

    def sbody(i, _):
        idx = cand_v[pl.ds(i * _LANES, _LANES)]
        plsc.store_scatter(memb_v, [idx], ones)
        return 0

    lax.fori_loop(0, chunks, sbody, 0)
    for g in range(src // _GRP):
        pltpu.sync_copy(memb_v.at[pl.ds(g * _GRP, _GRP)], sh_memb.at[g, s])


def _sc_hist_a_body(cand_hbm, out_hbm, cand_v, memb_v, red_v, psum_v,
                    sh_memb):
    h_cnt, _ = cand_hbm.shape
    c = lax.axis_index("c")
    s = lax.axis_index("s")

    @pl.when(jnp.logical_and(c == 0, s < h_cnt))
    def _():
        _memb_phase(cand_hbm, cand_v, memb_v, sh_memb, s)

    plsc.subcore_barrier()

    @pl.when(c == 0)
    def _():
        pltpu.sync_copy(sh_memb.at[s], red_v)
        for j in range(_GRP // _LANES):
            acc = red_v[0, pl.ds(j * _LANES, _LANES)]
            for h in range(1, h_cnt):
                acc = acc + red_v[h, pl.ds(j * _LANES, _LANES)]
            psum_v[pl.ds(j * _LANES, _LANES)] = acc
        pltpu.sync_copy(psum_v, out_hbm.at[s])


def _sc_hist_b_body(cand_hbm, hist_hbm, out_hbm, cand_v, memb_v, red_v,
                    ha_v, fin_v, pmax_v, res_v, sh_memb, sh_max):
    h_cnt, _ = cand_hbm.shape
    c = lax.axis_index("c")
    s = lax.axis_index("s")
    zeros = jnp.zeros((_LANES,), jnp.float32)

    @pl.when(jnp.logical_and(c == 0, s < h_cnt))
    def _():
        _memb_phase(cand_hbm, cand_v, memb_v, sh_memb, s)

    plsc.subcore_barrier()

    @pl.when(c == 0)
    def _():
        pltpu.sync_copy(sh_memb.at[s], red_v)
        pltpu.sync_copy(hist_hbm.at[s], ha_v)
        cm = zeros
        for j in range(_GRP // _LANES):
            acc = ha_v[pl.ds(j * _LANES, _LANES)]
            for h in range(h_cnt):
                acc = acc + red_v[h, pl.ds(j * _LANES, _LANES)]
            cm = jnp.maximum(cm, acc)
        pmax_v[...] = cm
        pltpu.sync_copy(pmax_v, sh_max.at[s])

    plsc.subcore_barrier()

    @pl.when(jnp.logical_and(c == 0, s == 0))
    def _():
        pltpu.sync_copy(sh_max, fin_v)
        m = fin_v[0]
        for t in range(1, fin_v.shape[0]):
            m = jnp.maximum(m, fin_v[t])
        fired = (jnp.max(m) <= _K).astype(jnp.int32)
        res_v[...] = jnp.full((_LANES,), fired, jnp.int32)
        pltpu.sync_copy(res_v, out_hbm)


def kernel(attention_weight):
    aw = attention_weight
    h_num, tgt, src = aw.shape
    rb_num = tgt // _ROWS
    ha = _HSPLIT
    hb = h_num - _HSPLIT
    groups = src // _GRP
    mesh = plsc.VectorSubcoreMesh(core_axis_name="c", subcore_axis_name="s")
    sc_params = pltpu.CompilerParams(needs_layout_passes=False)

    buf_a, cand_a = pl.pallas_call(
        _argmax_copy_body_a,
        grid=(ha, rb_num),
        in_specs=[pl.BlockSpec((1, _ROWS, src), lambda h, rb: (h, rb, 0))],
        out_specs=[
            pl.BlockSpec((1, _ROWS, src), lambda h, rb: (h, rb, 0)),
            pl.BlockSpec((1, 1, tgt), lambda h, rb: (h, 0, 0)),
        ],
        out_shape=[
            jax.ShapeDtypeStruct(aw.shape, aw.dtype),
            jax.ShapeDtypeStruct((ha, 1, tgt), jnp.int32),
        ],
    )(aw)

    buf_b, cand_b = pl.pallas_call(
        _argmax_copy_body_b,
        grid=(hb, rb_num),
        in_specs=[
            pl.BlockSpec((1, _ROWS, src), lambda h, rb: (h + _HSPLIT, rb, 0)),
            pl.BlockSpec(memory_space=pl.ANY),
        ],
        out_specs=[
            pl.BlockSpec((1, _ROWS, src), lambda h, rb: (h + _HSPLIT, rb, 0)),
            pl.BlockSpec((1, 1, tgt), lambda h, rb: (h, 0, 0)),
        ],
        out_shape=[
            jax.ShapeDtypeStruct(aw.shape, aw.dtype),
            jax.ShapeDtypeStruct((hb, 1, tgt), jnp.int32),
        ],
        input_output_aliases={1: 0},
    )(aw, buf_a)

    sc_a = pl.kernel(
        _sc_hist_a_body,
        out_type=jax.ShapeDtypeStruct((groups, _GRP), jnp.float32),
        mesh=mesh,
        compiler_params=sc_params,
        scratch_types=[
            pltpu.VMEM((tgt,), jnp.int32),
            pltpu.VMEM((src,), jnp.float32),
            pltpu.VMEM((ha, _GRP), jnp.float32),
            pltpu.VMEM((_GRP,), jnp.float32),
            pltpu.VMEM_SHARED((groups, ha, _GRP), jnp.float32),
        ],
    )
    hist_a = sc_a(cand_a.reshape(ha, tgt))

    sc_b = pl.kernel(
        _sc_hist_b_body,
        out_type=jax.ShapeDtypeStruct((_LANES,), jnp.int32),
        mesh=mesh,
        compiler_params=sc_params,
        scratch_types=[
            pltpu.VMEM((tgt,), jnp.int32),
            pltpu.VMEM((src,), jnp.float32),
            pltpu.VMEM((hb, _GRP), jnp.float32),
            pltpu.VMEM((_GRP,), jnp.float32),
            pltpu.VMEM((16, _LANES), jnp.float32),
            pltpu.VMEM((_LANES,), jnp.float32),
            pltpu.VMEM((_LANES,), jnp.int32),
            pltpu.VMEM_SHARED((groups, hb, _GRP), jnp.float32),
            pltpu.VMEM_SHARED((16, _LANES), jnp.float32),
        ],
    )
    cond = sc_b(cand_b.reshape(hb, tgt), hist_a)

    out = pl.pallas_call(
        _fixup_body,
        in_specs=[
            pl.BlockSpec(memory_space=pltpu.SMEM),
            pl.BlockSpec(memory_space=pl.ANY),
        ],
        out_specs=pl.BlockSpec(memory_space=pl.ANY),
        out_shape=jax.ShapeDtypeStruct(aw.shape, aw.dtype),
        scratch_shapes=[pltpu.SemaphoreType.DMA],
        input_output_aliases={1: 0},
    )(cond, buf_b)
    return out


# speedup vs baseline: 1.0016x; 1.0016x over previous
import jax
import jax.numpy as jnp
import numpy as np
from jax import lax
from jax.experimental import pallas as pl
from jax.experimental.pallas import tpu as pltpu
from jax.experimental.pallas import tpu_sc as plsc

_rng = np.random.default_rng(0)
_DRAWS = [int(_rng.integers(low=0, high=15)) for _ in range(4)]
_D0 = _DRAWS[0]
_K = 8.0

_ROWS = 1024
_HSPLIT = 12
_LANES = 16
_GRP = 128


def _argmax_copy_core(in_ref, out_ref, cand_ref):
    rb = pl.program_id(1)
    x = in_ref[0]
    out_ref[...] = in_ref[...]
    m = jnp.max(x, axis=1, keepdims=True)
    src = x.shape[1]
    iota = lax.broadcasted_iota(jnp.int32, x.shape, 1)
    idx = jnp.min(jnp.where(x == m, iota, src), axis=1)
    cand_ref[0, 0, pl.ds(rb * _ROWS, _ROWS)] = idx


def _argmax_copy_body_a(in_ref, out_ref, cand_ref):
    _argmax_copy_core(in_ref, out_ref, cand_ref)


def _argmax_copy_body_b(in_ref, buf_ref, out_ref, cand_ref):
    del buf_ref
    _argmax_copy_core(in_ref, out_ref, cand_ref)


def _fixup_body(cond_ref, buf_ref, out_ref, sem):
    del buf_ref

    @pl.when(cond_ref[0] == 1)
    def _():
        for h in range(out_ref.shape[0]):
            if h != _D0:
                cp = pltpu.make_async_copy(out_ref.at[_D0], out_ref.at[h], sem)
                cp.start()
                cp.wait()


def _memb_phase(cand_hbm, cand_v, memb_v, sh_memb, s):
    src = memb_v.shape[0]
    chunks = src // _LANES
    zeros = jnp.zeros((_LANES,), jnp.float32)
    ones = jnp.ones((_LANES,), jnp.float32)
    pltpu.sync_copy(cand_hbm.at[s], cand_v)

    def zbody(i, _):
        memb_v[pl.ds(i * _LANES, _LANES)] = zeros
        return 0

    lax.fori_loop(0, chunks, zbody, 0)

    def sbody(i, _):
        idx = cand_v[pl.ds(i * _LANES, _LANES)]
        plsc.store_scatter(memb_v, [idx], ones)
        return 0

    lax.fori_loop(0, chunks, sbody, 0)
    for g in range(src // _GRP):
        pltpu.sync_copy(memb_v.at[pl.ds(g * _GRP, _GRP)], sh_memb.at[g, s])


def _sc_hist_a_body(cand_hbm, out_hbm, cand_v, memb_v, red_v, psum_v,
                    sh_memb):
    h_cnt, _ = cand_hbm.shape
    c = lax.axis_index("c")
    s = lax.axis_index("s")

    @pl.when(jnp.logical_and(c == 0, s < h_cnt))
    def _():
        _memb_phase(cand_hbm, cand_v, memb_v, sh_memb, s)

    plsc.subcore_barrier()

    @pl.when(c == 0)
    def _():
        pltpu.sync_copy(sh_memb.at[s], red_v)
        for j in range(_GRP // _LANES):
            acc = red_v[0, pl.ds(j * _LANES, _LANES)]
            for h in range(1, h_cnt):
                acc = acc + red_v[h, pl.ds(j * _LANES, _LANES)]
            psum_v[pl.ds(j * _LANES, _LANES)] = acc
        pltpu.sync_copy(psum_v, out_hbm.at[s])


def _sc_hist_b_body(cand_hbm, hist_hbm, out_hbm, cand_v, memb_v, red_v,
                    ha_v, fin_v, pmax_v, res_v, sh_memb, sh_max):
    h_cnt, _ = cand_hbm.shape
    c = lax.axis_index("c")
    s = lax.axis_index("s")
    zeros = jnp.zeros((_LANES,), jnp.float32)

    @pl.when(jnp.logical_and(c == 0, s < h_cnt))
    def _():
        _memb_phase(cand_hbm, cand_v, memb_v, sh_memb, s)

    plsc.subcore_barrier()

    @pl.when(c == 0)
    def _():
        pltpu.sync_copy(sh_memb.at[s], red_v)
        pltpu.sync_copy(hist_hbm.at[s], ha_v)
        cm = zeros
        for j in range(_GRP // _LANES):
            acc = ha_v[pl.ds(j * _LANES, _LANES)]
            for h in range(h_cnt):
                acc = acc + red_v[h, pl.ds(j * _LANES, _LANES)]
            cm = jnp.maximum(cm, acc)
        pmax_v[...] = cm
        pltpu.sync_copy(pmax_v, sh_max.at[s])

    plsc.subcore_barrier()

    @pl.when(jnp.logical_and(c == 0, s == 0))
    def _():
        pltpu.sync_copy(sh_max, fin_v)
        m = fin_v[0]
        for t in range(1, fin_v.shape[0]):
            m = jnp.maximum(m, fin_v[t])
        fired = (jnp.max(m) <= _K).astype(jnp.int32)
        res_v[...] = jnp.full((_LANES,), fired, jnp.int32)
        pltpu.sync_copy(res_v, out_hbm)


def kernel(attention_weight):
    aw = attention_weight
    h_num, tgt, src = aw.shape
    rb_num = tgt // _ROWS
    ha = _HSPLIT
    hb = h_num - _HSPLIT
    groups = src // _GRP
    mesh = plsc.VectorSubcoreMesh(core_axis_name="c", subcore_axis_name="s")
    sc_params = pltpu.CompilerParams(needs_layout_passes=False)

    buf_a, cand_a = pl.pallas_call(
        _argmax_copy_body_a,
        grid=(ha, rb_num),
        in_specs=[pl.BlockSpec((1, _ROWS, src), lambda h, rb: (h, rb, 0))],
        out_specs=[
            pl.BlockSpec((1, _ROWS, src), lambda h, rb: (h, rb, 0)),
            pl.BlockSpec((1, 1, tgt), lambda h, rb: (h, 0, 0)),
        ],
        out_shape=[
            jax.ShapeDtypeStruct(aw.shape, aw.dtype),
            jax.ShapeDtypeStruct((ha, 1, tgt), jnp.int32),
        ],
    )(aw)

    sc_a = pl.kernel(
        _sc_hist_a_body,
        out_type=jax.ShapeDtypeStruct((groups, _GRP), jnp.float32),
        mesh=mesh,
        compiler_params=sc_params,
        scratch_types=[
            pltpu.VMEM((tgt,), jnp.int32),
            pltpu.VMEM((src,), jnp.float32),
            pltpu.VMEM((ha, _GRP), jnp.float32),
            pltpu.VMEM((_GRP,), jnp.float32),
            pltpu.VMEM_SHARED((groups, ha, _GRP), jnp.float32),
        ],
    )
    hist_a = sc_a(cand_a.reshape(ha, tgt))

    buf_b, cand_b = pl.pallas_call(
        _argmax_copy_body_b,
        grid=(hb, rb_num),
        in_specs=[
            pl.BlockSpec((1, _ROWS, src), lambda h, rb: (h + _HSPLIT, rb, 0)),
            pl.BlockSpec(memory_space=pl.ANY),
        ],
        out_specs=[
            pl.BlockSpec((1, _ROWS, src), lambda h, rb: (h + _HSPLIT, rb, 0)),
            pl.BlockSpec((1, 1, tgt), lambda h, rb: (h, 0, 0)),
        ],
        out_shape=[
            jax.ShapeDtypeStruct(aw.shape, aw.dtype),
            jax.ShapeDtypeStruct((hb, 1, tgt), jnp.int32),
        ],
        input_output_aliases={1: 0},
    )(aw, buf_a)

    sc_b = pl.kernel(
        _sc_hist_b_body,
        out_type=jax.ShapeDtypeStruct((_LANES,), jnp.int32),
        mesh=mesh,
        compiler_params=sc_params,
        scratch_types=[
            pltpu.VMEM((tgt,), jnp.int32),
            pltpu.VMEM((src,), jnp.float32),
            pltpu.VMEM((hb, _GRP), jnp.float32),
            pltpu.VMEM((_GRP,), jnp.float32),
            pltpu.VMEM((16, _LANES), jnp.float32),
            pltpu.VMEM((_LANES,), jnp.float32),
            pltpu.VMEM((_LANES,), jnp.int32),
            pltpu.VMEM_SHARED((groups, hb, _GRP), jnp.float32),
            pltpu.VMEM_SHARED((16, _LANES), jnp.float32),
        ],
    )
    cond = sc_b(cand_b.reshape(hb, tgt), hist_a)

    out = pl.pallas_call(
        _fixup_body,
        in_specs=[
            pl.BlockSpec(memory_space=pltpu.SMEM),
            pl.BlockSpec(memory_space=pl.ANY),
        ],
        out_specs=pl.BlockSpec(memory_space=pl.ANY),
        out_shape=jax.ShapeDtypeStruct(aw.shape, aw.dtype),
        scratch_shapes=[pltpu.SemaphoreType.DMA],
        input_output_aliases={1: 0},
    )(cond, buf_b)
    return out


# speedup vs baseline: 1.0183x; 1.0167x over previous
import jax
import jax.numpy as jnp
import numpy as np
from jax import lax
from jax.experimental import pallas as pl
from jax.experimental.pallas import tpu as pltpu
from jax.experimental.pallas import tpu_sc as plsc

_rng = np.random.default_rng(0)
_DRAWS = [int(_rng.integers(low=0, high=15)) for _ in range(4)]
_D0 = _DRAWS[0]
_K = 8.0

_ROWS = 1024


def _argmax_copy_body(in_ref, out_ref, cand_ref):
    rb = pl.program_id(1)
    x = in_ref[0]
    out_ref[...] = in_ref[...]
    m = jnp.max(x, axis=1, keepdims=True)
    src = x.shape[1]
    iota = lax.broadcasted_iota(jnp.int32, x.shape, 1)
    idx = jnp.min(jnp.where(x == m, iota, src), axis=1)
    cand_ref[0, 0, pl.ds(rb * _ROWS, _ROWS)] = idx


def _fixup_body(cond_ref, buf_ref, out_ref, sem):
    del buf_ref

    @pl.when(cond_ref[0] == 1)
    def _():
        for h in range(out_ref.shape[0]):
            if h != _D0:
                cp = pltpu.make_async_copy(out_ref.at[_D0], out_ref.at[h], sem)
                cp.start()
                cp.wait()


_LANES = 16


def _sc_hist_body(cand_hbm, out_hbm, cand_v, memb_v, red_v, fin_v, pmax_v,
                  res_v, sh_memb, sh_max):
    h_num, src = cand_hbm.shape
    chunks = src // _LANES
    groups = src // 128
    c = lax.axis_index("c")
    s = lax.axis_index("s")
    zeros = jnp.zeros((_LANES,), jnp.float32)
    ones = jnp.ones((_LANES,), jnp.float32)

    @pl.when(c == 0)
    def _():
        pltpu.sync_copy(cand_hbm.at[s], cand_v)

        def zbody(i, _):
            memb_v[pl.ds(i * _LANES, _LANES)] = zeros
            return 0

        lax.fori_loop(0, chunks, zbody, 0)

        def sbody(i, _):
            idx = cand_v[pl.ds(i * _LANES, _LANES)]
            plsc.store_scatter(memb_v, [idx], ones)
            return 0

        lax.fori_loop(0, chunks, sbody, 0)
        for g in range(groups):
            pltpu.sync_copy(memb_v.at[pl.ds(g * 128, 128)], sh_memb.at[g, s])

    plsc.subcore_barrier()

    @pl.when(c == 0)
    def _():
        pltpu.sync_copy(sh_memb.at[s], red_v)
        cm = zeros
        for j in range(128 // _LANES):
            acc = red_v[0, pl.ds(j * _LANES, _LANES)]
            for h in range(1, h_num):
                acc = acc + red_v[h, pl.ds(j * _LANES, _LANES)]
            cm = jnp.maximum(cm, acc)
        pmax_v[...] = cm
        pltpu.sync_copy(pmax_v, sh_max.at[s])

    plsc.subcore_barrier()

    @pl.when(jnp.logical_and(c == 0, s == 0))
    def _():
        pltpu.sync_copy(sh_max, fin_v)
        m = fin_v[0]
        for t in range(1, fin_v.shape[0]):
            m = jnp.maximum(m, fin_v[t])
        fired = (jnp.max(m) <= _K).astype(jnp.int32)
        res_v[...] = jnp.full((_LANES,), fired, jnp.int32)
        pltpu.sync_copy(res_v, out_hbm)


def kernel(attention_weight):
    aw = attention_weight
    h_num, tgt, src = aw.shape
    rb_num = tgt // _ROWS

    copy_out, cand = pl.pallas_call(
        _argmax_copy_body,
        grid=(h_num, rb_num),
        in_specs=[pl.BlockSpec((1, _ROWS, src), lambda h, rb: (h, rb, 0))],
        out_specs=[
            pl.BlockSpec((1, _ROWS, src), lambda h, rb: (h, rb, 0)),
            pl.BlockSpec((1, 1, tgt), lambda h, rb: (h, 0, 0)),
        ],
        out_shape=[
            jax.ShapeDtypeStruct(aw.shape, aw.dtype),
            jax.ShapeDtypeStruct((h_num, 1, tgt), jnp.int32),
        ],
    )(aw)

    cand2 = cand.reshape(h_num, tgt)
    sc_hist = pl.kernel(
        _sc_hist_body,
        out_type=jax.ShapeDtypeStruct((_LANES,), jnp.int32),
        mesh=plsc.VectorSubcoreMesh(core_axis_name="c", subcore_axis_name="s"),
        compiler_params=pltpu.CompilerParams(needs_layout_passes=False),
        scratch_types=[
            pltpu.VMEM((tgt,), jnp.int32),
            pltpu.VMEM((src,), jnp.float32),
            pltpu.VMEM((h_num, 128), jnp.float32),
            pltpu.VMEM((16, _LANES), jnp.float32),
            pltpu.VMEM((_LANES,), jnp.float32),
            pltpu.VMEM((_LANES,), jnp.int32),
            pltpu.VMEM_SHARED((src // 128, h_num, 128), jnp.float32),
            pltpu.VMEM_SHARED((16, _LANES), jnp.float32),
        ],
    )
    cond = sc_hist(cand2)

    out = pl.pallas_call(
        _fixup_body,
        in_specs=[
            pl.BlockSpec(memory_space=pltpu.SMEM),
            pl.BlockSpec(memory_space=pl.ANY),
        ],
        out_specs=pl.BlockSpec(memory_space=pl.ANY),
        out_shape=jax.ShapeDtypeStruct(aw.shape, aw.dtype),
        scratch_shapes=[pltpu.SemaphoreType.DMA],
        input_output_aliases={1: 0},
    )(cond, copy_out)
    return out
